# Initial kernel scaffold; baseline (speedup 1.0000x reference)
import functools

import jax
import jax.numpy as jnp
from jax import lax
from jax.experimental import pallas as pl
from jax.experimental.pallas import tpu as pltpu
from jax.experimental.pallas import tpu_sc as plsc

K = 16
M1, M2 = 20, 10
NC, NS = 2, 16
NW = NC * NS
CB = 128


def _fm_body(idx1_hbm, idx2_hbm, v1_hbm, v2_hbm, out_hbm,
             idx1_v, idx2_v, rows1_v, rows2_v, out_v, sem1, sem2,
             *, batch):
    per_w = batch // NW
    nblk = per_w // CB
    wid = lax.axis_index("s") * NC + lax.axis_index("c")
    base = wid * per_w

    def block(t, _):
        eb = base + t * CB
        pltpu.sync_copy(idx1_hbm.at[pl.ds(eb * M1, CB * M1)], idx1_v)
        pltpu.sync_copy(idx2_hbm.at[pl.ds(eb * M2, CB * M2)], idx2_v)
        cp1 = pltpu.async_copy(v1_hbm.at[idx1_v], rows1_v, sem1)
        cp2 = pltpu.async_copy(v2_hbm.at[idx2_v], rows2_v, sem2)
        cp1.wait()
        cp2.wait()

        def elem(i, _):
            s = jnp.zeros((K,), jnp.float32)
            q = jnp.zeros((K,), jnp.float32)
            for j in range(M1):
                n = i * M1 + j
                m = jnp.where(idx1_v[n] != 0, 1.0, 0.0).astype(jnp.float32)
                r = rows1_v[n]
                rm = r * m
                s = s + rm
                q = q + rm * r
            for j in range(M2):
                n = i * M2 + j
                m = jnp.where(idx2_v[n] != 0, 1.0, 0.0).astype(jnp.float32)
                r = rows2_v[n]
                rm = r * m
                s = s + rm
                q = q + rm * r
            out_v[i] = 0.5 * jnp.sum(s * s - q)
            return _

        lax.fori_loop(0, CB, elem, None)
        pltpu.sync_copy(out_v, out_hbm.at[pl.ds(eb, CB)])
        return _

    lax.fori_loop(0, nblk, block, None)


def kernel(idx1, idx2, V1, V2):
    batch = idx1.shape[0]
    mesh = plsc.VectorSubcoreMesh(
        core_axis_name="c", subcore_axis_name="s",
        num_cores=NC, num_subcores=NS)
    run = pl.kernel(
        functools.partial(_fm_body, batch=batch),
        out_type=jax.ShapeDtypeStruct((batch,), jnp.float32),
        mesh=mesh,
        scratch_types=[
            pltpu.VMEM((CB * M1,), jnp.int32),
            pltpu.VMEM((CB * M2,), jnp.int32),
            pltpu.VMEM((CB * M1, K), jnp.float32),
            pltpu.VMEM((CB * M2, K), jnp.float32),
            pltpu.VMEM((CB,), jnp.float32),
            pltpu.SemaphoreType.DMA,
            pltpu.SemaphoreType.DMA,
        ],
    )
    return run(idx1.reshape(-1), idx2.reshape(-1), V1, V2)


# speedup vs baseline: 1.7934x; 1.7934x over previous
import functools

import jax
import jax.numpy as jnp
from jax import lax
from jax.experimental import pallas as pl
from jax.experimental.pallas import tpu as pltpu
from jax.experimental.pallas import tpu_sc as plsc

K = 16
M1, M2 = 20, 10
NC, NS = 2, 16
NW = NC * NS
CB = 128


def _fm_body(idx1_hbm, idx2_hbm, v1_hbm, v2_hbm, out_hbm,
             idx1_v, idx2_v, rows1_v, rows2_v, out_v, sem1, sem2,
             *, batch):
    per_w = batch // NW
    nblk = per_w // CB
    wid = lax.axis_index("s") * NC + lax.axis_index("c")
    base = wid * per_w

    def block(t, _):
        eb = base + t * CB
        pltpu.sync_copy(idx1_hbm.at[pl.ds(eb * M1, CB * M1)], idx1_v)
        pltpu.sync_copy(idx2_hbm.at[pl.ds(eb * M2, CB * M2)], idx2_v)
        cp1 = pltpu.async_copy(v1_hbm.at[idx1_v], rows1_v, sem1)
        cp2 = pltpu.async_copy(v2_hbm.at[idx2_v], rows2_v, sem2)
        cp1.wait()
        cp2.wait()

        lane0 = lax.iota(jnp.int32, K) == 0

        def elem(i, _):
            s = jnp.zeros((K,), jnp.float32)
            q = jnp.zeros((K,), jnp.float32)
            for j in range(M1):
                n = i * M1 + j
                iv = plsc.load_gather(idx1_v, [jnp.full((K,), n, jnp.int32)])
                m = jnp.where(iv != 0, 1.0, 0.0)
                r = rows1_v[n]
                rm = r * m
                s = s + rm
                q = q + rm * r
            for j in range(M2):
                n = i * M2 + j
                iv = plsc.load_gather(idx2_v, [jnp.full((K,), n, jnp.int32)])
                m = jnp.where(iv != 0, 1.0, 0.0)
                r = rows2_v[n]
                rm = r * m
                s = s + rm
                q = q + rm * r
            red = 0.5 * jnp.sum(s * s - q)
            plsc.store_scatter(out_v, [jnp.full((K,), i, jnp.int32)],
                               jnp.broadcast_to(red, (K,)), mask=lane0)
            return _

        lax.fori_loop(0, CB, elem, None)
        pltpu.sync_copy(out_v, out_hbm.at[pl.ds(eb, CB)])
        return _

    lax.fori_loop(0, nblk, block, None)


def kernel(idx1, idx2, V1, V2):
    batch = idx1.shape[0]
    mesh = plsc.VectorSubcoreMesh(
        core_axis_name="c", subcore_axis_name="s",
        num_cores=NC, num_subcores=NS)
    run = pl.kernel(
        functools.partial(_fm_body, batch=batch),
        out_type=jax.ShapeDtypeStruct((batch,), jnp.float32),
        mesh=mesh,
        scratch_types=[
            pltpu.VMEM((CB * M1,), jnp.int32),
            pltpu.VMEM((CB * M2,), jnp.int32),
            pltpu.VMEM((CB * M1, K), jnp.float32),
            pltpu.VMEM((CB * M2, K), jnp.float32),
            pltpu.VMEM((CB,), jnp.float32),
            pltpu.SemaphoreType.DMA,
            pltpu.SemaphoreType.DMA,
        ],
        compiler_params=pltpu.CompilerParams(
            needs_layout_passes=False, use_tc_tiling_on_sc=False),
    )
    return run(idx1.reshape(-1), idx2.reshape(-1), V1, V2)
